# baseline (speedup 1.0000x reference)
import functools

import jax
import jax.numpy as jnp
from jax import lax
from jax.experimental import pallas as pl
from jax.experimental.pallas import tpu as pltpu
from jax.experimental.pallas import tpu_sc as plsc

_NC = 2
_NS = 16
_NW = _NC * _NS
_CHUNK = 128
_NBUF = 2
_GROUP = 16
_BSHIFT = 3
_NBKT = 1280


def _sc_sort(ep, epw, srcp, dstp):
    mesh = plsc.VectorSubcoreMesh(core_axis_name="c", subcore_axis_name="s")

    @functools.partial(
        pl.kernel,
        out_type=(
            jax.ShapeDtypeStruct((ep,), jnp.int32),
            jax.ShapeDtypeStruct((ep,), jnp.int32),
        ),
        mesh=mesh,
        compiler_params=pltpu.CompilerParams(needs_layout_passes=False),
        scratch_types=[
            pltpu.VMEM((epw,), jnp.int32),
            pltpu.VMEM((epw,), jnp.int32),
            pltpu.VMEM((epw,), jnp.int32),
            pltpu.VMEM((epw,), jnp.int32),
            pltpu.VMEM((_NBKT * 16,), jnp.int32),
        ],
    )
    def srt(src_hbm, dst_hbm, src_o, dst_o, src_f, dst_f, src_t, dst_t, cnt):
        core = lax.axis_index("c")
        sub = lax.axis_index("s")
        wid = core * _NS + sub
        ebase = wid * epw

        pltpu.sync_copy(src_hbm.at[pl.ds(ebase, epw)], src_f)
        pltpu.sync_copy(dst_hbm.at[pl.ds(ebase, epw)], dst_f)

        def zero_cnt(i, c2):
            cnt[pl.ds(i * 16, 16)] = jnp.zeros((16,), jnp.int32)
            return c2

        lax.fori_loop(0, _NBKT, zero_cnt, None)

        def count_vec(i, c2):
            lanes = lax.iota(jnp.int32, 16)
            s = src_f[pl.ds(i * 16, 16)]
            slot = lax.shift_left(
                lax.shift_right_logical(s, _BSHIFT), 4) + lanes
            cur = plsc.load_gather(cnt, [slot])
            plsc.store_scatter(cnt, [slot], cur + 1)
            return c2

        lax.fori_loop(0, epw // 16, count_vec, None)

        def scan_vec(i, carry_in):
            cr = carry_in
            for c in range(8):
                v = cnt[pl.ds(i * 128 + c * 16, 16)]
                ex = plsc.cumsum(v) - v
                cnt[pl.ds(i * 128 + c * 16, 16)] = ex + cr
                cr = cr + jnp.sum(v)
            return cr

        lax.fori_loop(0, _NBKT * 16 // 128, scan_vec, jnp.int32(0))

        def place_vec(i, c2):
            lanes = lax.iota(jnp.int32, 16)
            s = src_f[pl.ds(i * 16, 16)]
            t = dst_f[pl.ds(i * 16, 16)]
            slot = lax.shift_left(
                lax.shift_right_logical(s, _BSHIFT), 4) + lanes
            pos = plsc.load_gather(cnt, [slot])
            plsc.store_scatter(cnt, [slot], pos + 1)
            plsc.store_scatter(src_t, [pos], s)
            plsc.store_scatter(dst_t, [pos], t)
            return c2

        lax.fori_loop(0, epw // 16, place_vec, None)

        pltpu.sync_copy(src_t, src_o.at[pl.ds(ebase, epw)])
        pltpu.sync_copy(dst_t, dst_o.at[pl.ds(ebase, epw)])

    return srt(srcp, dstp)


def _sc_aggregate(n_pad, c0, c1, x, srcp, dstp, zeros):
    d = x.shape[1]
    rows_pt = n_pad // _NS
    mesh = plsc.VectorSubcoreMesh(core_axis_name="c", subcore_axis_name="s")

    @functools.partial(
        pl.kernel,
        out_type=(
            jax.ShapeDtypeStruct((n_pad, d), jnp.float32),
            jax.ShapeDtypeStruct((n_pad, d), jnp.float32),
        ),
        mesh=mesh,
        scratch_types=[
            pltpu.VMEM((_GROUP, _CHUNK), jnp.int32),
            pltpu.VMEM((_GROUP, _CHUNK), jnp.int32),
            [pltpu.VMEM((_CHUNK, d), jnp.float32)] * _NBUF,
            pltpu.VMEM_SHARED((n_pad, d), jnp.float32),
            [pltpu.SemaphoreType.DMA] * _NBUF,
        ],
    )
    def agg(x_hbm, src_hbm, dst_hbm, zero_hbm, out0, out1,
            src_i, dst_i, rows, acc, sems):
        core = lax.axis_index("c")
        sub = lax.axis_index("s")
        rbase = sub * rows_pt
        cpw = jnp.where(core == 0, c0, c1)
        tbase = jnp.where(core == 0, sub * c0, _NS * c0 + sub * c1)

        pltpu.sync_copy(zero_hbm.at[pl.ds(rbase, rows_pt)],
                        acc.at[pl.ds(rbase, rows_pt)])
        plsc.subcore_barrier()

        def group_body(gr, carry):
            cbase = tbase + gr * _GROUP
            pltpu.sync_copy(src_hbm.at[pl.ds(cbase, _GROUP)], src_i)
            pltpu.sync_copy(dst_hbm.at[pl.ds(cbase, _GROUP)], dst_i)

            for b in range(_NBUF):
                pltpu.async_copy(x_hbm.at[src_i.at[b]], rows[b], sems[b])

            def body(jo, c2):
                for b in range(_NBUF):
                    j = jo * _NBUF + b
                    pltpu.make_async_copy(
                        x_hbm.at[src_i.at[j]], rows[b], sems[b]).wait()
                    pltpu.sync_copy(rows[b], acc.at[dst_i.at[j]], add=True)

                    @pl.when(j + _NBUF < _GROUP)
                    def _():
                        pltpu.async_copy(
                            x_hbm.at[src_i.at[j + _NBUF]], rows[b], sems[b])

                return c2

            lax.fori_loop(0, _GROUP // _NBUF, body, None)
            return carry

        lax.fori_loop(0, cpw // _GROUP, group_body, None)
        plsc.subcore_barrier()


        @pl.when(core == 0)
        def _():
            pltpu.sync_copy(acc.at[pl.ds(rbase, rows_pt)],
                            out0.at[pl.ds(rbase, rows_pt)])

        @pl.when(core == 1)
        def _():
            pltpu.sync_copy(acc.at[pl.ds(rbase, rows_pt)],
                            out1.at[pl.ds(rbase, rows_pt)])

    return agg(x, srcp, dstp, zeros)


def _tc_matmul(p0, p1, w, b, n):
    d_in, d_out = w.shape
    blk = 400
    grid = n // blk

    def mm(p0_ref, p1_ref, w_ref, b_ref, o_ref):
        a = p0_ref[...] + p1_ref[...]
        o_ref[...] = (
            jnp.dot(a, w_ref[...], preferred_element_type=jnp.float32)
            + b_ref[...]
        )

    return pl.pallas_call(
        mm,
        grid=(grid,),
        in_specs=[
            pl.BlockSpec((blk, d_in), lambda i: (i, 0)),
            pl.BlockSpec((blk, d_in), lambda i: (i, 0)),
            pl.BlockSpec((d_in, d_out), lambda i: (0, 0)),
            pl.BlockSpec((1, d_out), lambda i: (0, 0)),
        ],
        out_specs=pl.BlockSpec((blk, d_out), lambda i: (i, 0)),
        out_shape=jax.ShapeDtypeStruct((n, d_out), jnp.float32),
    )(p0, p1, w, b)


def kernel(x, edge_index, W, b):
    n, d = x.shape
    e = edge_index.shape[1]

    tpt = -(-e // (_NS * _CHUNK))
    c0 = max(_GROUP, int(round(tpt * 1.0 / _GROUP)) * _GROUP)
    c1 = -(-(tpt - c0) // _GROUP) * _GROUP
    ep = _NS * (c0 + c1) * _CHUNK
    rows_pt = -(-(n + 1) // _NS)
    rows_pt = -(-rows_pt // 8) * 8
    n_pad = rows_pt * _NS

    src = edge_index[0]
    dst = edge_index[1]
    pad = ep - e
    srcp = jnp.concatenate([src, jnp.zeros((pad,), jnp.int32)])
    dstp = jnp.concatenate([dst, jnp.full((pad,), n, jnp.int32)])
    srcp = srcp.reshape(ep // _CHUNK, _CHUNK)
    dstp = dstp.reshape(ep // _CHUNK, _CHUNK)
    zeros = jnp.zeros((n_pad, d), jnp.float32)

    p0, p1 = _sc_aggregate(n_pad, c0, c1, x, srcp, dstp, zeros)
    return _tc_matmul(p0, p1, W, b, n)


# speedup vs baseline: 1.4402x; 1.4402x over previous
import functools

import jax
import jax.numpy as jnp
from jax import lax
from jax.experimental import pallas as pl
from jax.experimental.pallas import tpu as pltpu
from jax.experimental.pallas import tpu_sc as plsc

_NC = 2
_NS = 16
_NW = _NC * _NS
_CHUNK = 128
_NBUF = 2
_GROUP = 8
_BSHIFT = 3
_NBKT = 1280


def _sc_sort(ep, epw, srcp, dstp):
    mesh = plsc.VectorSubcoreMesh(core_axis_name="c", subcore_axis_name="s")

    @functools.partial(
        pl.kernel,
        out_type=(
            jax.ShapeDtypeStruct((ep,), jnp.int32),
            jax.ShapeDtypeStruct((ep,), jnp.int32),
        ),
        mesh=mesh,
        compiler_params=pltpu.CompilerParams(needs_layout_passes=False),
        scratch_types=[
            pltpu.VMEM((epw,), jnp.int32),
            pltpu.VMEM((epw,), jnp.int32),
            pltpu.VMEM((epw,), jnp.int32),
            pltpu.VMEM((epw,), jnp.int32),
            pltpu.VMEM((_NBKT * 16,), jnp.int32),
        ],
    )
    def srt(src_hbm, dst_hbm, src_o, dst_o, src_f, dst_f, src_t, dst_t, cnt):
        core = lax.axis_index("c")
        sub = lax.axis_index("s")
        wid = core * _NS + sub
        ebase = wid * epw

        pltpu.sync_copy(src_hbm.at[pl.ds(ebase, epw)], src_f)
        pltpu.sync_copy(dst_hbm.at[pl.ds(ebase, epw)], dst_f)

        def zero_cnt(i, c2):
            cnt[pl.ds(i * 16, 16)] = jnp.zeros((16,), jnp.int32)
            return c2

        lax.fori_loop(0, _NBKT, zero_cnt, None)

        def count_vec(i, c2):
            lanes = lax.iota(jnp.int32, 16)
            s = src_f[pl.ds(i * 16, 16)]
            slot = lax.shift_left(
                lax.shift_right_logical(s, _BSHIFT), 4) + lanes
            cur = plsc.load_gather(cnt, [slot])
            plsc.store_scatter(cnt, [slot], cur + 1)
            return c2

        lax.fori_loop(0, epw // 16, count_vec, None)

        def scan_vec(i, carry_in):
            cr = carry_in
            for c in range(8):
                v = cnt[pl.ds(i * 128 + c * 16, 16)]
                ex = plsc.cumsum(v) - v
                cnt[pl.ds(i * 128 + c * 16, 16)] = ex + cr
                cr = cr + jnp.sum(v)
            return cr

        lax.fori_loop(0, _NBKT * 16 // 128, scan_vec, jnp.int32(0))

        def place_vec(i, c2):
            lanes = lax.iota(jnp.int32, 16)
            s = src_f[pl.ds(i * 16, 16)]
            t = dst_f[pl.ds(i * 16, 16)]
            slot = lax.shift_left(
                lax.shift_right_logical(s, _BSHIFT), 4) + lanes
            pos = plsc.load_gather(cnt, [slot])
            plsc.store_scatter(cnt, [slot], pos + 1)
            plsc.store_scatter(src_t, [pos], s)
            plsc.store_scatter(dst_t, [pos], t)
            return c2

        lax.fori_loop(0, epw // 16, place_vec, None)

        pltpu.sync_copy(src_t, src_o.at[pl.ds(ebase, epw)])
        pltpu.sync_copy(dst_t, dst_o.at[pl.ds(ebase, epw)])

    return srt(srcp, dstp)


def _sc_aggregate(n_pad, c0, c1, x, srcp, dstp, zeros):
    d = x.shape[1]
    rows_pt = n_pad // _NS
    mesh = plsc.VectorSubcoreMesh(core_axis_name="c", subcore_axis_name="s")

    @functools.partial(
        pl.kernel,
        out_type=(
            jax.ShapeDtypeStruct((n_pad, d), jnp.float32),
            jax.ShapeDtypeStruct((n_pad, d), jnp.float32),
        ),
        mesh=mesh,
        scratch_types=[
            pltpu.VMEM((_GROUP, _CHUNK), jnp.int32),
            pltpu.VMEM((_GROUP, _CHUNK), jnp.int32),
            [pltpu.VMEM((_CHUNK, d), jnp.float32)] * _NBUF,
            pltpu.VMEM_SHARED((n_pad, d), jnp.float32),
            [pltpu.SemaphoreType.DMA] * _NBUF,
        ],
    )
    def agg(x_hbm, src_hbm, dst_hbm, zero_hbm, out0, out1,
            src_i, dst_i, rows, acc, sems):
        core = lax.axis_index("c")
        sub = lax.axis_index("s")
        rbase = sub * rows_pt
        cpw = jnp.where(core == 0, c0, c1)
        tbase = jnp.where(core == 0, sub * c0, _NS * c0 + sub * c1)

        pltpu.sync_copy(zero_hbm.at[pl.ds(rbase, rows_pt)],
                        acc.at[pl.ds(rbase, rows_pt)])
        plsc.subcore_barrier()

        def group_body(gr, carry):
            cbase = tbase + gr * _GROUP
            pltpu.sync_copy(src_hbm.at[pl.ds(cbase, _GROUP)], src_i)
            pltpu.sync_copy(dst_hbm.at[pl.ds(cbase, _GROUP)], dst_i)

            for b in range(_NBUF):
                pltpu.async_copy(x_hbm.at[src_i.at[b]], rows[b], sems[b])

            def body(jo, c2):
                for b in range(_NBUF):
                    j = jo * _NBUF + b
                    pltpu.make_async_copy(
                        x_hbm.at[src_i.at[j]], rows[b], sems[b]).wait()
                    pltpu.sync_copy(rows[b], acc.at[dst_i.at[j]], add=True)

                    @pl.when(j + _NBUF < _GROUP)
                    def _():
                        pltpu.async_copy(
                            x_hbm.at[src_i.at[j + _NBUF]], rows[b], sems[b])

                return c2

            lax.fori_loop(0, _GROUP // _NBUF, body, None)
            return carry

        lax.fori_loop(0, cpw // _GROUP, group_body, None)
        plsc.subcore_barrier()


        @pl.when(core == 0)
        def _():
            pltpu.sync_copy(acc.at[pl.ds(rbase, rows_pt)],
                            out0.at[pl.ds(rbase, rows_pt)])

        @pl.when(core == 1)
        def _():
            pltpu.sync_copy(acc.at[pl.ds(rbase, rows_pt)],
                            out1.at[pl.ds(rbase, rows_pt)])

    return agg(x, srcp, dstp, zeros)


def _tc_matmul(p0, p1, w, b, n):
    d_in, d_out = w.shape
    blk = 400
    grid = n // blk

    def mm(p0_ref, p1_ref, w_ref, b_ref, o_ref):
        a = p0_ref[...] + p1_ref[...]
        o_ref[...] = (
            jnp.dot(a, w_ref[...], preferred_element_type=jnp.float32)
            + b_ref[...]
        )

    return pl.pallas_call(
        mm,
        grid=(grid,),
        in_specs=[
            pl.BlockSpec((blk, d_in), lambda i: (i, 0)),
            pl.BlockSpec((blk, d_in), lambda i: (i, 0)),
            pl.BlockSpec((d_in, d_out), lambda i: (0, 0)),
            pl.BlockSpec((1, d_out), lambda i: (0, 0)),
        ],
        out_specs=pl.BlockSpec((blk, d_out), lambda i: (i, 0)),
        out_shape=jax.ShapeDtypeStruct((n, d_out), jnp.float32),
    )(p0, p1, w, b)


def kernel(x, edge_index, W, b):
    n, d = x.shape
    e = edge_index.shape[1]

    tpt = -(-e // (_NS * _CHUNK))
    c0 = max(_GROUP, int(round(tpt * 0.95 / _GROUP)) * _GROUP)
    c1 = -(-(tpt - c0) // _GROUP) * _GROUP
    ep = _NS * (c0 + c1) * _CHUNK
    rows_pt = -(-(n + 1) // _NS)
    rows_pt = -(-rows_pt // 8) * 8
    n_pad = rows_pt * _NS

    src = edge_index[0]
    dst = edge_index[1]
    pad = ep - e
    srcp = jnp.concatenate([src, jnp.zeros((pad,), jnp.int32)])
    dstp = jnp.concatenate([dst, jnp.full((pad,), n, jnp.int32)])
    srcp = srcp.reshape(ep // _CHUNK, _CHUNK)
    dstp = dstp.reshape(ep // _CHUNK, _CHUNK)
    zeros = jnp.zeros((n_pad, d), jnp.float32)

    p0, p1 = _sc_aggregate(n_pad, c0, c1, x, srcp, dstp, zeros)
    return _tc_matmul(p0, p1, W, b, n)
